# Initial kernel scaffold; baseline (speedup 1.0000x reference)
import jax
import jax.numpy as jnp
from jax.experimental import pallas as pl


def kernel(write_weights, free_gate, read_weights, prev_usage):
    raise NotImplementedError("write your pallas kernel here")


# speedup vs baseline: 2.5674x; 2.5674x over previous
import jax
import jax.numpy as jnp
from jax.experimental import pallas as pl
from jax.experimental.pallas import tpu as pltpu

B = 1024
M = 16384
BB = 256
BM = 2048


def _body(fg_ref, ww_ref, rw_ref, pu_ref, out_ref):
    w0 = ww_ref[:, 0, :]
    w1 = ww_ref[:, 1, :]
    ww = 1.0 - (1.0 - w0) * (1.0 - w1)
    pu = pu_ref[...]
    usage = pu + (1.0 - pu) * ww
    fg = fg_ref[...]
    phi = 1.0 - fg[:, 0][:, None] * rw_ref[:, 0, :]
    for r in range(1, 4):
        phi = phi * (1.0 - fg[:, r][:, None] * rw_ref[:, r, :])
    out_ref[...] = jnp.clip(usage * phi, 0.0, 1.0)


def kernel(write_weights, free_gate, read_weights, prev_usage):
    grid = (B // BB, M // BM)
    return pl.pallas_call(
        _body,
        grid=grid,
        in_specs=[
            pl.BlockSpec((BB, 4), lambda i, j: (i, 0)),
            pl.BlockSpec((BB, 2, BM), lambda i, j: (i, 0, j)),
            pl.BlockSpec((BB, 4, BM), lambda i, j: (i, 0, j)),
            pl.BlockSpec((BB, BM), lambda i, j: (i, j)),
        ],
        out_specs=pl.BlockSpec((BB, BM), lambda i, j: (i, j)),
        out_shape=jax.ShapeDtypeStruct((B, M), jnp.float32),
        compiler_params=pltpu.CompilerParams(
            dimension_semantics=("arbitrary", "arbitrary"),
        ),
    )(free_gate, write_weights, read_weights, prev_usage)


# speedup vs baseline: 2.6110x; 1.0170x over previous
import jax
import jax.numpy as jnp
from jax.experimental import pallas as pl
from jax.experimental.pallas import tpu as pltpu

B = 1024
M = 16384
BB = 128
BM = 4096


def _body(fg_ref, ww_ref, rw_ref, pu_ref, out_ref):
    w0 = ww_ref[:, 0, :]
    w1 = ww_ref[:, 1, :]
    ww = 1.0 - (1.0 - w0) * (1.0 - w1)
    pu = pu_ref[...]
    usage = pu + (1.0 - pu) * ww
    fg = fg_ref[...]
    phi = 1.0 - fg[:, 0][:, None] * rw_ref[:, 0, :]
    for r in range(1, 4):
        phi = phi * (1.0 - fg[:, r][:, None] * rw_ref[:, r, :])
    out_ref[...] = jnp.clip(usage * phi, 0.0, 1.0)


def kernel(write_weights, free_gate, read_weights, prev_usage):
    grid = (B // BB, M // BM)
    return pl.pallas_call(
        _body,
        grid=grid,
        in_specs=[
            pl.BlockSpec((BB, 4), lambda i, j: (i, 0)),
            pl.BlockSpec((BB, 2, BM), lambda i, j: (i, 0, j)),
            pl.BlockSpec((BB, 4, BM), lambda i, j: (i, 0, j)),
            pl.BlockSpec((BB, BM), lambda i, j: (i, j)),
        ],
        out_specs=pl.BlockSpec((BB, BM), lambda i, j: (i, j)),
        out_shape=jax.ShapeDtypeStruct((B, M), jnp.float32),
        compiler_params=pltpu.CompilerParams(
            dimension_semantics=("arbitrary", "arbitrary"),
        ),
    )(free_gate, write_weights, read_weights, prev_usage)
